# initial kernel scaffold (unmeasured)
import functools

import jax
import jax.numpy as jnp
from jax import lax
from jax.experimental import pallas as pl
from jax.experimental.pallas import tpu as pltpu

N_DEV = 8
B_PER = 128
D = 128
H_PER = 256


def kernel(x, Win0, Wout0, Win1, Wout1, Win2, Wout2):
    def body(
        x_ref, win0_ref, wout0_ref, win1_ref, wout1_ref, win2_ref, wout2_ref,
        out_ref,
        wg_in0, wg_out0, wg_in1, wg_out1, wg_in2, wg_out2,
        send_sems, recv_sems,
    ):
        my = lax.axis_index("i")

        in_refs = [win0_ref, win1_ref, win2_ref]
        out_refs = [wout0_ref, wout1_ref, wout2_ref]
        wg_ins = [wg_in0, wg_in1, wg_in2]
        wg_outs = [wg_out0, wg_out1, wg_out2]

        barrier = pltpu.get_barrier_semaphore()
        for d in range(1, N_DEV):
            pl.semaphore_signal(
                barrier, inc=1,
                device_id=(lax.rem(my + d, N_DEV),),
                device_id_type=pl.DeviceIdType.MESH,
            )
        pl.semaphore_wait(barrier, N_DEV - 1)

        for k in range(3):
            wg_ins[k][my] = in_refs[k][...].astype(jnp.bfloat16)
            wg_outs[k][my] = out_refs[k][...].astype(jnp.bfloat16)

        for k in range(3):
            for wg, tsem in ((wg_ins[k], 2 * k), (wg_outs[k], 2 * k + 1)):
                for d in range(1, N_DEV):
                    p = lax.rem(my + d, N_DEV)
                    rdma = pltpu.make_async_remote_copy(
                        src_ref=wg.at[my],
                        dst_ref=wg.at[my],
                        send_sem=send_sems.at[tsem],
                        recv_sem=recv_sems.at[tsem],
                        device_id=(p,),
                        device_id_type=pl.DeviceIdType.MESH,
                    )
                    rdma.start()

        acts = x_ref[...].astype(jnp.bfloat16)
        acc = jnp.zeros((B_PER, D), jnp.float32)
        for k in range(3):
            pl.semaphore_wait(recv_sems.at[2 * k], N_DEV - 1)
            pl.semaphore_wait(recv_sems.at[2 * k + 1], N_DEV - 1)
            acc = jnp.zeros((B_PER, D), jnp.float32)
            for p in range(N_DEV):
                h = jnp.dot(acts, wg_ins[k][p],
                            preferred_element_type=jnp.float32)
                h = jnp.maximum(h, 0.0).astype(jnp.bfloat16)
                acc = acc + jnp.dot(h, wg_outs[k][p],
                                    preferred_element_type=jnp.float32)
            acts = acc.astype(jnp.bfloat16)
        out_ref[...] = acc

        for t in range(6):
            pl.semaphore_wait(send_sems.at[t], N_DEV - 1)

        @functools.partial(pl.run_scoped, exit_sem=pltpu.SemaphoreType.REGULAR)
        def _(exit_sem):
            for d in range(1, N_DEV):
                pl.semaphore_signal(
                    exit_sem, inc=1,
                    device_id=(lax.rem(my + d, N_DEV),),
                    device_id_type=pl.DeviceIdType.MESH,
                )
            pl.semaphore_wait(exit_sem, N_DEV - 1)

    return pl.pallas_call(
        body,
        out_shape=jax.ShapeDtypeStruct((B_PER, D), jnp.float32),
        in_specs=[pl.BlockSpec(memory_space=pltpu.VMEM)] * 7,
        out_specs=pl.BlockSpec(memory_space=pltpu.VMEM),
        scratch_shapes=[
            pltpu.VMEM((N_DEV, D, H_PER), jnp.bfloat16),
            pltpu.VMEM((N_DEV, H_PER, D), jnp.bfloat16),
            pltpu.VMEM((N_DEV, D, H_PER), jnp.bfloat16),
            pltpu.VMEM((N_DEV, H_PER, D), jnp.bfloat16),
            pltpu.VMEM((N_DEV, D, H_PER), jnp.bfloat16),
            pltpu.VMEM((N_DEV, H_PER, D), jnp.bfloat16),
            pltpu.SemaphoreType.DMA((6,)),
            pltpu.SemaphoreType.DMA((6,)),
        ],
        compiler_params=pltpu.CompilerParams(collective_id=0),
    )(x, Win0, Wout0, Win1, Wout1, Win2, Wout2)


# baseline (device time: 33757 ns/iter reference)
import functools

import jax
import jax.numpy as jnp
from jax import lax
from jax.experimental import pallas as pl
from jax.experimental.pallas import tpu as pltpu

N_DEV = 8
B_PER = 128
D = 128
H_PER = 256


def kernel(x, Win0, Wout0, Win1, Wout1, Win2, Wout2):
    def body(
        x_ref, win0_ref, wout0_ref, win1_ref, wout1_ref, win2_ref, wout2_ref,
        out_ref,
        wg_in0, wg_out0, wg_in1, wg_out1, wg_in2, wg_out2,
        send_sems, recv_sems,
    ):
        my = lax.axis_index("i")

        in_refs = [win0_ref, win1_ref, win2_ref]
        out_refs = [wout0_ref, wout1_ref, wout2_ref]
        wg_ins = [wg_in0, wg_in1, wg_in2]
        wg_outs = [wg_out0, wg_out1, wg_out2]

        barrier = pltpu.get_barrier_semaphore()
        for d in range(1, N_DEV):
            pl.semaphore_signal(
                barrier, inc=1,
                device_id=(lax.rem(my + d, N_DEV),),
                device_id_type=pl.DeviceIdType.MESH,
            )
        pl.semaphore_wait(barrier, N_DEV - 1)

        for k in range(3):
            wg_ins[k][my] = in_refs[k][...].astype(jnp.bfloat16)
            wg_outs[k][my] = out_refs[k][...].astype(jnp.bfloat16)

        rdmas = [[] for _ in range(6)]
        for k in range(3):
            for wg, tsem in ((wg_ins[k], 2 * k), (wg_outs[k], 2 * k + 1)):
                for d in range(1, N_DEV):
                    p = lax.rem(my + d, N_DEV)
                    rdma = pltpu.make_async_remote_copy(
                        src_ref=wg.at[my],
                        dst_ref=wg.at[my],
                        send_sem=send_sems.at[tsem],
                        recv_sem=recv_sems.at[tsem],
                        device_id=(p,),
                        device_id_type=pl.DeviceIdType.MESH,
                    )
                    rdma.start()
                    rdmas[tsem].append(rdma)

        acts = x_ref[...].astype(jnp.bfloat16)
        acc = jnp.zeros((B_PER, D), jnp.float32)
        for k in range(3):
            for rdma in rdmas[2 * k]:
                rdma.wait_recv()
            for rdma in rdmas[2 * k + 1]:
                rdma.wait_recv()
            acc = jnp.zeros((B_PER, D), jnp.float32)
            for p in range(N_DEV):
                h = jnp.dot(acts, wg_ins[k][p],
                            preferred_element_type=jnp.float32)
                h = jnp.maximum(h, 0.0).astype(jnp.bfloat16)
                acc = acc + jnp.dot(h, wg_outs[k][p],
                                    preferred_element_type=jnp.float32)
            acts = acc.astype(jnp.bfloat16)
        out_ref[...] = acc

        for t in range(6):
            for rdma in rdmas[t]:
                rdma.wait_send()

        @functools.partial(pl.run_scoped, exit_sem=pltpu.SemaphoreType.REGULAR)
        def _(exit_sem):
            for d in range(1, N_DEV):
                pl.semaphore_signal(
                    exit_sem, inc=1,
                    device_id=(lax.rem(my + d, N_DEV),),
                    device_id_type=pl.DeviceIdType.MESH,
                )
            pl.semaphore_wait(exit_sem, N_DEV - 1)

    return pl.pallas_call(
        body,
        out_shape=jax.ShapeDtypeStruct((B_PER, D), jnp.float32),
        in_specs=[pl.BlockSpec(memory_space=pltpu.VMEM)] * 7,
        out_specs=pl.BlockSpec(memory_space=pltpu.VMEM),
        scratch_shapes=[
            pltpu.VMEM((N_DEV, D, H_PER), jnp.bfloat16),
            pltpu.VMEM((N_DEV, H_PER, D), jnp.bfloat16),
            pltpu.VMEM((N_DEV, D, H_PER), jnp.bfloat16),
            pltpu.VMEM((N_DEV, H_PER, D), jnp.bfloat16),
            pltpu.VMEM((N_DEV, D, H_PER), jnp.bfloat16),
            pltpu.VMEM((N_DEV, H_PER, D), jnp.bfloat16),
            pltpu.SemaphoreType.DMA((6,)),
            pltpu.SemaphoreType.DMA((6,)),
        ],
        compiler_params=pltpu.CompilerParams(collective_id=0),
    )(x, Win0, Wout0, Win1, Wout1, Win2, Wout2)


# device time: 33554 ns/iter; 1.0060x vs baseline; 1.0060x over previous
import functools

import jax
import jax.numpy as jnp
from jax import lax
from jax.experimental import pallas as pl
from jax.experimental.pallas import tpu as pltpu

N_DEV = 8
B_PER = 128
D = 128
H_PER = 256
H = N_DEV * H_PER


def kernel(x, Win0, Wout0, Win1, Wout1, Win2, Wout2):
    def body(
        x_ref, win0_ref, wout0_ref, win1_ref, wout1_ref, win2_ref, wout2_ref,
        out_ref,
        wg_in0, wg_out0, wg_in1, wg_out1, wg_in2, wg_out2,
        send_sems, recv_sems,
    ):
        my = lax.axis_index("i")

        in_refs = [win0_ref, win1_ref, win2_ref]
        out_refs = [wout0_ref, wout1_ref, wout2_ref]
        wg_ins = [wg_in0, wg_in1, wg_in2]
        wg_outs = [wg_out0, wg_out1, wg_out2]

        barrier = pltpu.get_barrier_semaphore()
        for d in range(1, N_DEV):
            pl.semaphore_signal(
                barrier, inc=1,
                device_id=(lax.rem(my + d, N_DEV),),
                device_id_type=pl.DeviceIdType.MESH,
            )
        pl.semaphore_wait(barrier, N_DEV - 1)

        col = pl.ds(my * H_PER, H_PER)
        for k in range(3):
            wg_ins[k][:, col] = in_refs[k][...].astype(jnp.bfloat16)
            wg_outs[k][col, :] = out_refs[k][...].astype(jnp.bfloat16)

        rdmas = [[] for _ in range(6)]
        for k in range(3):
            for wg, slc, tsem in (
                (wg_ins[k], (slice(None), col), 2 * k),
                (wg_outs[k], (col, slice(None)), 2 * k + 1),
            ):
                for d in range(1, N_DEV):
                    p = lax.rem(my + d, N_DEV)
                    rdma = pltpu.make_async_remote_copy(
                        src_ref=wg.at[slc],
                        dst_ref=wg.at[slc],
                        send_sem=send_sems.at[tsem],
                        recv_sem=recv_sems.at[tsem],
                        device_id=(p,),
                        device_id_type=pl.DeviceIdType.MESH,
                    )
                    rdma.start()
                    rdmas[tsem].append(rdma)

        acts = x_ref[...].astype(jnp.bfloat16)
        acc = jnp.zeros((B_PER, D), jnp.float32)
        for k in range(3):
            for rdma in rdmas[2 * k]:
                rdma.wait_recv()
            for rdma in rdmas[2 * k + 1]:
                rdma.wait_recv()
            h = jnp.dot(acts, wg_ins[k][...],
                        preferred_element_type=jnp.float32)
            h = jnp.maximum(h, 0.0).astype(jnp.bfloat16)
            acc = jnp.dot(h, wg_outs[k][...],
                          preferred_element_type=jnp.float32)
            acts = acc.astype(jnp.bfloat16)
        out_ref[...] = acc

        for t in range(6):
            for rdma in rdmas[t]:
                rdma.wait_send()

        @functools.partial(pl.run_scoped, exit_sem=pltpu.SemaphoreType.REGULAR)
        def _(exit_sem):
            for d in range(1, N_DEV):
                pl.semaphore_signal(
                    exit_sem, inc=1,
                    device_id=(lax.rem(my + d, N_DEV),),
                    device_id_type=pl.DeviceIdType.MESH,
                )
            pl.semaphore_wait(exit_sem, N_DEV - 1)

    return pl.pallas_call(
        body,
        out_shape=jax.ShapeDtypeStruct((B_PER, D), jnp.float32),
        in_specs=[pl.BlockSpec(memory_space=pltpu.VMEM)] * 7,
        out_specs=pl.BlockSpec(memory_space=pltpu.VMEM),
        scratch_shapes=[
            pltpu.VMEM((D, H), jnp.bfloat16),
            pltpu.VMEM((H, D), jnp.bfloat16),
            pltpu.VMEM((D, H), jnp.bfloat16),
            pltpu.VMEM((H, D), jnp.bfloat16),
            pltpu.VMEM((D, H), jnp.bfloat16),
            pltpu.VMEM((H, D), jnp.bfloat16),
            pltpu.SemaphoreType.DMA((6,)),
            pltpu.SemaphoreType.DMA((6,)),
        ],
        compiler_params=pltpu.CompilerParams(collective_id=0),
    )(x, Win0, Wout0, Win1, Wout1, Win2, Wout2)


# device time: 32032 ns/iter; 1.0539x vs baseline; 1.0475x over previous
import functools

import jax
import jax.numpy as jnp
from jax import lax
from jax.experimental import pallas as pl
from jax.experimental.pallas import tpu as pltpu

N_DEV = 8
PLANE = 4
B_PER = 128
D = 128
H_PER = 256
B_GRP = PLANE * B_PER


def kernel(x, Win0, Wout0, Win1, Wout1, Win2, Wout2):
    def body(
        x_ref, win0_ref, wout0_ref, win1_ref, wout1_ref, win2_ref, wout2_ref,
        out_ref,
        wins, wouts, xg, prtl0, prtl1, prtl2, racc0, racc1, racc2, xn0, xn1,
        w_send, w_recv, a_send, a_recv,
    ):
        my = lax.axis_index("i")
        q = lax.rem(my, PLANE)
        base = my - q
        partner = lax.rem(my + PLANE, N_DEV)

        in_refs = [win0_ref, win1_ref, win2_ref]
        out_refs = [wout0_ref, wout1_ref, wout2_ref]
        prtls = [prtl0, prtl1, prtl2]
        raccs = [racc0, racc1, racc2]
        xns = [xn0, xn1]

        def my_peers():
            yield partner
            for dq in range(1, PLANE):
                yield base + lax.rem(q + dq, PLANE)

        barrier = pltpu.get_barrier_semaphore()
        for p in my_peers():
            pl.semaphore_signal(
                barrier, inc=1,
                device_id=(p,),
                device_id_type=pl.DeviceIdType.MESH,
            )
        pl.semaphore_wait(barrier, PLANE)

        for k in range(3):
            wins[0, k] = in_refs[k][...].astype(jnp.bfloat16)
            wouts[0, k] = out_refs[k][...].astype(jnp.bfloat16)
        xg[q] = x_ref[...].astype(jnp.bfloat16)

        w_rdmas = []
        for k in range(3):
            for wg, tsem in ((wins, 2 * k), (wouts, 2 * k + 1)):
                rdma = pltpu.make_async_remote_copy(
                    src_ref=wg.at[0, k],
                    dst_ref=wg.at[1, k],
                    send_sem=w_send.at[tsem],
                    recv_sem=w_recv.at[tsem],
                    device_id=(partner,),
                    device_id_type=pl.DeviceIdType.MESH,
                )
                rdma.start()
                w_rdmas.append(rdma)

        def plane_bcast(src_ref_slot, dst_ref_slot, psem):
            rds = []
            for dq in range(1, PLANE):
                p = base + lax.rem(q + dq, PLANE)
                rdma = pltpu.make_async_remote_copy(
                    src_ref=src_ref_slot,
                    dst_ref=dst_ref_slot,
                    send_sem=a_send.at[psem],
                    recv_sem=a_recv.at[psem],
                    device_id=(p,),
                    device_id_type=pl.DeviceIdType.MESH,
                )
                rdma.start()
                rds.append(rdma)
            return rds

        xag_rdmas = plane_bcast(xg.at[q], xg.at[q], 0)
        for rdma in xag_rdmas:
            rdma.wait_recv()
        acts = jnp.reshape(xg[...], (B_GRP, D))

        all_rdmas = list(w_rdmas) + xag_rdmas
        for k in range(3):
            w_rdmas[2 * k].wait_recv()
            w_rdmas[2 * k + 1].wait_recv()
            partial = jnp.zeros((B_GRP, D), jnp.float32)
            for s in range(2):
                h = jnp.dot(acts, wins[s, k],
                            preferred_element_type=jnp.float32)
                h = jnp.maximum(h, 0.0).astype(jnp.bfloat16)
                partial = partial + jnp.dot(h, wouts[s, k],
                                            preferred_element_type=jnp.float32)
            prtls[k][...] = jnp.reshape(
                partial.astype(jnp.bfloat16), (PLANE, B_PER, D))
            rs_rdmas = []
            for dq in range(1, PLANE):
                p = base + lax.rem(q + dq, PLANE)
                qp = lax.rem(q + dq, PLANE)
                rdma = pltpu.make_async_remote_copy(
                    src_ref=prtls[k].at[qp],
                    dst_ref=raccs[k].at[q],
                    send_sem=a_send.at[1 + 2 * k],
                    recv_sem=a_recv.at[1 + 2 * k],
                    device_id=(p,),
                    device_id_type=pl.DeviceIdType.MESH,
                )
                rdma.start()
                rs_rdmas.append(rdma)
            for rdma in rs_rdmas:
                rdma.wait_recv()
            all_rdmas.extend(rs_rdmas)
            reduced = prtls[k][q].astype(jnp.float32)
            for dq in range(1, PLANE):
                qp = lax.rem(q + dq, PLANE)
                reduced = reduced + raccs[k][qp].astype(jnp.float32)
            if k < 2:
                xns[k][q] = reduced.astype(jnp.bfloat16)
                ag_rdmas = plane_bcast(xns[k].at[q], xns[k].at[q], 2 + 2 * k)
                for rdma in ag_rdmas:
                    rdma.wait_recv()
                all_rdmas.extend(ag_rdmas)
                acts = jnp.reshape(xns[k][...], (B_GRP, D))
            else:
                out_ref[...] = reduced

        for rdma in all_rdmas:
            rdma.wait_send()

        @functools.partial(pl.run_scoped, exit_sem=pltpu.SemaphoreType.REGULAR)
        def _(exit_sem):
            for p in my_peers():
                pl.semaphore_signal(
                    exit_sem, inc=1,
                    device_id=(p,),
                    device_id_type=pl.DeviceIdType.MESH,
                )
            pl.semaphore_wait(exit_sem, PLANE)

    return pl.pallas_call(
        body,
        out_shape=jax.ShapeDtypeStruct((B_PER, D), jnp.float32),
        in_specs=[pl.BlockSpec(memory_space=pltpu.VMEM)] * 7,
        out_specs=pl.BlockSpec(memory_space=pltpu.VMEM),
        scratch_shapes=[
            pltpu.VMEM((2, 3, D, H_PER), jnp.bfloat16),
            pltpu.VMEM((2, 3, H_PER, D), jnp.bfloat16),
            pltpu.VMEM((PLANE, B_PER, D), jnp.bfloat16),
            pltpu.VMEM((PLANE, B_PER, D), jnp.bfloat16),
            pltpu.VMEM((PLANE, B_PER, D), jnp.bfloat16),
            pltpu.VMEM((PLANE, B_PER, D), jnp.bfloat16),
            pltpu.VMEM((PLANE, B_PER, D), jnp.bfloat16),
            pltpu.VMEM((PLANE, B_PER, D), jnp.bfloat16),
            pltpu.VMEM((PLANE, B_PER, D), jnp.bfloat16),
            pltpu.VMEM((PLANE, B_PER, D), jnp.bfloat16),
            pltpu.VMEM((PLANE, B_PER, D), jnp.bfloat16),
            pltpu.SemaphoreType.DMA((6,)),
            pltpu.SemaphoreType.DMA((6,)),
            pltpu.SemaphoreType.DMA((6,)),
            pltpu.SemaphoreType.DMA((6,)),
        ],
        compiler_params=pltpu.CompilerParams(collective_id=0),
    )(x, Win0, Wout0, Win1, Wout1, Win2, Wout2)
